# baseline (speedup 1.0000x reference)
import functools

import jax
import jax.numpy as jnp
from jax import lax
from jax.experimental import pallas as pl
from jax.experimental.pallas import tpu as pltpu
from jax.experimental.pallas import tpu_sc as plsc

N = 50000
D = 64
E = 800000
NUM_LAYERS = 3

NC, NS, LANES = 2, 16, 16

HALF = 25088
NPAD = 2 * HALF
ROWS_PER_TILE = HALF // NS
DUMMY = HALF
ACC_ROWS = 25600
ZROWS_PER_TILE = ACC_ROWS // NS

CHUNK = 128
BLK = 512
RB = BLK // CHUNK
NB = 98
NBLK = NS * NB
EPAD = NBLK * BLK

ZCH = 80
OCH = 112

_mesh = plsc.VectorSubcoreMesh(
    core_axis_name="c", subcore_axis_name="s", num_cores=NC, num_subcores=NS
)
_sc_params = pltpu.CompilerParams(use_tc_tiling_on_sc=False)


def _remap_rows(ib, base):
    for r in range(RB):
        for i in range(CHUNK // LANES):
            sl = (r, pl.ds(i * LANES, LANES))
            v = ib[sl]
            ok = jnp.logical_and(v >= base, v < base + HALF)
            ib[sl] = jnp.where(ok, v - base, DUMMY)


def _zero_acc(acc, obuf, sid, width):
    @pl.loop(0, ZCH)
    def _(i):
        for j in range(width // LANES):
            obuf[i, pl.ds(j * LANES, LANES)] = jnp.zeros((LANES,), jnp.float32)

    @pl.loop(0, ZROWS_PER_TILE // ZCH)
    def _(j):
        pltpu.sync_copy(
            obuf.at[pl.ds(0, ZCH)],
            acc.at[pl.ds(sid * ZROWS_PER_TILE + j * ZCH, ZCH)])


@jax.jit
def _deg_scale(idxc):
    @functools.partial(
        pl.kernel,
        out_type=jax.ShapeDtypeStruct((NPAD, LANES), jnp.float32),
        mesh=_mesh,
        compiler_params=_sc_params,
        scratch_types=[
            pltpu.VMEM_SHARED((ACC_ROWS, LANES), jnp.float32),
            pltpu.VMEM((RB * 2, CHUNK), jnp.int32),
            pltpu.VMEM((RB * 2, CHUNK), jnp.int32),
            pltpu.VMEM((CHUNK, LANES), jnp.float32),
            pltpu.VMEM((OCH, LANES), jnp.float32),
            pltpu.SemaphoreType.DMA,
            pltpu.SemaphoreType.DMA,
            pltpu.SemaphoreType.DMA,
        ],
    )
    def k(idx_hbm, scale_hbm, acc, iba, ibb, ones, obuf, sema, semb, sems):
        cid = lax.axis_index("c")
        sid = lax.axis_index("s")
        base = cid * HALF
        b0 = sid * NB

        @pl.loop(0, CHUNK)
        def _(i):
            ones[i, :] = jnp.ones((LANES,), jnp.float32)

        _zero_acc(acc, obuf, sid, LANES)
        plsc.subcore_barrier()

        pltpu.async_copy(idx_hbm.at[b0], iba, sema)
        pltpu.async_copy(idx_hbm.at[b0 + 1], ibb, semb)

        @pl.loop(0, NB // 2)
        def _(j):
            blk = b0 + 2 * j
            for ib, sem, off in ((iba, sema, 2), (ibb, semb, 3)):
                pltpu.make_async_copy(idx_hbm.at[b0], ib, sem).wait()
                _remap_rows(ib, base)
                hs = [pltpu.async_copy(ones, acc.at[ib.at[r]], sems, add=True)
                      for r in range(RB)]
                for h in hs:
                    h.wait()

                @pl.when(j < NB // 2 - 1)
                def _():
                    pltpu.async_copy(idx_hbm.at[blk + off], ib, sem)

        plsc.subcore_barrier()

        @pl.loop(0, ROWS_PER_TILE // OCH)
        def _(kk):
            r0 = sid * ROWS_PER_TILE + kk * OCH
            pltpu.sync_copy(acc.at[pl.ds(r0, OCH)], obuf)

            @pl.loop(0, OCH)
            def _(i):
                d = obuf[i, :]
                obuf[i, :] = jnp.where(d > 0.0, 1.0 / d, 0.0)

            pltpu.sync_copy(obuf, scale_hbm.at[pl.ds(base + r0, OCH)])

    return k(idxc)


DH = D // 2
ACC2_ROWS = NPAD + 8
DUMMY2 = NPAD
NROWS_PER_TILE = NPAD // NS


@jax.jit
def _layer(xs, idxc, scale):
    @functools.partial(
        pl.kernel,
        out_type=jax.ShapeDtypeStruct((2, NPAD, DH), jnp.float32),
        mesh=_mesh,
        compiler_params=_sc_params,
        scratch_types=[
            pltpu.VMEM_SHARED((ACC2_ROWS, DH), jnp.float32),
            pltpu.VMEM((RB * 2, CHUNK), jnp.int32),
            pltpu.VMEM((RB * 2, CHUNK), jnp.int32),
            pltpu.VMEM((CHUNK, DH), jnp.float32),
            pltpu.VMEM((CHUNK, DH), jnp.float32),
            pltpu.VMEM((CHUNK, DH), jnp.float32),
            pltpu.VMEM((CHUNK, DH), jnp.float32),
            pltpu.VMEM((OCH, DH), jnp.float32),
            pltpu.VMEM((OCH, LANES), jnp.float32),
            pltpu.SemaphoreType.DMA,
            pltpu.SemaphoreType.DMA,
            [pltpu.SemaphoreType.DMA] * 4,
            [pltpu.SemaphoreType.DMA] * 4,
        ],
    )
    def k(x_hbm, idx_hbm, scale_hbm, xn_hbm,
          acc, iba, ibb, xb0, xb1, xb2, xb3, obuf, sbuf,
          sema, semb, gs, ss):
        cid = lax.axis_index("c")
        sid = lax.axis_index("s")
        b0 = sid * NB
        xbs = (xb0, xb1, xb2, xb3)

        _zero_acc2(acc, obuf, sid)
        plsc.subcore_barrier()

        pltpu.async_copy(idx_hbm.at[b0], iba, sema)
        pltpu.async_copy(idx_hbm.at[b0 + 1], ibb, semb)

        @pl.loop(0, NB // 2)
        def _(j):
            blk = b0 + 2 * j
            pltpu.make_async_copy(idx_hbm.at[b0], iba, sema).wait()
            gA = [pltpu.async_copy(x_hbm.at[cid].at[iba.at[RB + r]],
                                   xbs[r], gs[r]) for r in range(RB)]
            sA = []
            for r in range(RB):
                gA[r].wait()
                sA.append(pltpu.async_copy(
                    xbs[r], acc.at[iba.at[r]], ss[r], add=True))

            pltpu.make_async_copy(idx_hbm.at[b0], ibb, semb).wait()
            gB = []
            for r in range(RB):
                sA[r].wait()
                gB.append(pltpu.async_copy(
                    x_hbm.at[cid].at[ibb.at[RB + r]], xbs[r], gs[r]))

            @pl.when(j < NB // 2 - 1)
            def _():
                pltpu.async_copy(idx_hbm.at[blk + 2], iba, sema)
            sB = []
            for r in range(RB):
                gB[r].wait()
                sB.append(pltpu.async_copy(
                    xbs[r], acc.at[ibb.at[r]], ss[r], add=True))
            for h in sB:
                h.wait()

            @pl.when(j < NB // 2 - 1)
            def _():
                pltpu.async_copy(idx_hbm.at[blk + 3], ibb, semb)

        plsc.subcore_barrier()

        @pl.loop(0, NROWS_PER_TILE // OCH)
        def _(kk):
            r0 = sid * NROWS_PER_TILE + kk * OCH
            pltpu.sync_copy(acc.at[pl.ds(r0, OCH)], obuf)
            pltpu.sync_copy(scale_hbm.at[pl.ds(r0, OCH)], sbuf)

            @pl.loop(0, OCH)
            def _(i):
                s = sbuf[i, :]
                for j in range(DH // LANES):
                    sl = pl.ds(j * LANES, LANES)
                    obuf[i, sl] = obuf[i, sl] * s

            pltpu.sync_copy(obuf, xn_hbm.at[cid].at[pl.ds(r0, OCH)])

    return k(xs, idxc, scale)


def _zero_acc2(acc, obuf, sid):
    @pl.loop(0, OCH)
    def _(i):
        for j in range(DH // LANES):
            obuf[i, pl.ds(j * LANES, LANES)] = jnp.zeros((LANES,), jnp.float32)

    @pl.loop(0, NROWS_PER_TILE // OCH)
    def _(j):
        pltpu.sync_copy(
            obuf, acc.at[pl.ds(sid * NROWS_PER_TILE + j * OCH, OCH)])


def _mean4_kernel(a_ref, b_ref, c_ref, d_ref, o_ref):
    o_ref[...] = (a_ref[...] + b_ref[...] + c_ref[...] + d_ref[...]) * 0.25


@jax.jit
def _mean4(a, b, c, d):
    blk = 3136
    spec = pl.BlockSpec((1, blk, DH), lambda i, j: (i, j, 0))
    return pl.pallas_call(
        _mean4_kernel,
        grid=(2, NPAD // blk),
        in_specs=[spec, spec, spec, spec],
        out_specs=spec,
        out_shape=jax.ShapeDtypeStruct((2, NPAD, DH), jnp.float32),
    )(a, b, c, d)


@jax.jit
def kernel(edge_index, weight):
    row = edge_index[0].astype(jnp.int32)
    col = edge_index[1].astype(jnp.int32)
    rowp = jnp.concatenate([row, jnp.full((EPAD - E,), DUMMY2, jnp.int32)])
    colp = jnp.concatenate([col, jnp.zeros((EPAD - E,), jnp.int32)])
    idxc = jnp.concatenate(
        [rowp.reshape(NBLK, RB, CHUNK), colp.reshape(NBLK, RB, CHUNK)], axis=1)
    pad = jnp.zeros((NPAD - N, DH), jnp.float32)
    x0 = jnp.stack([jnp.concatenate([weight[:, :DH], pad]),
                    jnp.concatenate([weight[:, DH:], pad])])

    scale = _deg_scale(idxc)
    x1 = _layer(x0, idxc, scale)
    x2 = _layer(x1, idxc, scale)
    x3 = _layer(x2, idxc, scale)
    m = _mean4(x0, x1, x2, x3)
    return jnp.concatenate([m[0], m[1]], axis=1)[:N]


# speedup vs baseline: 1.2162x; 1.2162x over previous
import functools

import jax
import jax.numpy as jnp
from jax import lax
from jax.experimental import pallas as pl
from jax.experimental.pallas import tpu as pltpu
from jax.experimental.pallas import tpu_sc as plsc

N = 50000
D = 64
E = 800000
NUM_LAYERS = 3

NC, NS, LANES = 2, 16, 16

NPAD = 50176

CHUNK = 128
BLK = 512
RB = BLK // CHUNK
NB = 98
NBLK = NS * NB
EPAD = NBLK * BLK

OCH = 112

_mesh = plsc.VectorSubcoreMesh(
    core_axis_name="c", subcore_axis_name="s", num_cores=NC, num_subcores=NS
)
_sc_params = pltpu.CompilerParams(use_tc_tiling_on_sc=False)


@jax.jit
def _deg_partial(idxc):
    @functools.partial(
        pl.kernel,
        out_type=jax.ShapeDtypeStruct((2, NPAD, LANES), jnp.float32),
        mesh=_mesh,
        compiler_params=_sc_params,
        scratch_types=[
            pltpu.VMEM_SHARED((ACC2_ROWS, LANES), jnp.float32),
            pltpu.VMEM((RB * 2, CHUNK), jnp.int32),
            pltpu.VMEM((RB * 2, CHUNK), jnp.int32),
            pltpu.VMEM((CHUNK, LANES), jnp.float32),
            pltpu.VMEM((OCH, LANES), jnp.float32),
            pltpu.SemaphoreType.DMA,
            pltpu.SemaphoreType.DMA,
            pltpu.SemaphoreType.DMA,
        ],
    )
    def k(idx_hbm, deg_hbm, acc, iba, ibb, ones, obuf, sema, semb, sems):
        cid = lax.axis_index("c")
        sid = lax.axis_index("s")
        b0 = sid * NB

        @pl.loop(0, CHUNK)
        def _(i):
            ones[i, :] = jnp.ones((LANES,), jnp.float32)

        @pl.loop(0, OCH)
        def _(i):
            obuf[i, :] = jnp.zeros((LANES,), jnp.float32)

        @pl.loop(0, NROWS_PER_TILE // OCH)
        def _(j):
            pltpu.sync_copy(
                obuf, acc.at[pl.ds(sid * NROWS_PER_TILE + j * OCH, OCH)])

        plsc.subcore_barrier()

        pltpu.async_copy(idx_hbm.at[b0], iba, sema)
        pltpu.async_copy(idx_hbm.at[b0 + 1], ibb, semb)

        @pl.loop(0, NB // 2)
        def _(j):
            blk = b0 + 2 * j
            for ib, sem, off in ((iba, sema, 2), (ibb, semb, 3)):
                pltpu.make_async_copy(idx_hbm.at[b0], ib, sem).wait()
                hs = [pltpu.async_copy(
                          ones, acc.at[ib.at[2 * cid + r]], sems, add=True)
                      for r in range(2)]
                for h in hs:
                    h.wait()

                @pl.when(j < NB // 2 - 1)
                def _():
                    pltpu.async_copy(idx_hbm.at[blk + off], ib, sem)

        plsc.subcore_barrier()

        @pl.loop(0, NROWS_PER_TILE // OCH)
        def _(kk):
            r0 = sid * NROWS_PER_TILE + kk * OCH
            pltpu.sync_copy(acc.at[pl.ds(r0, OCH)],
                            deg_hbm.at[cid].at[pl.ds(r0, OCH)])

    return k(idxc)


DH = D // 2
ACC2_ROWS = NPAD + 8
DUMMY2 = NPAD
NROWS_PER_TILE = NPAD // NS


@jax.jit
def _layer(xs, idxc, degp):
    @functools.partial(
        pl.kernel,
        out_type=jax.ShapeDtypeStruct((2, NPAD, DH), jnp.float32),
        mesh=_mesh,
        compiler_params=_sc_params,
        scratch_types=[
            pltpu.VMEM_SHARED((ACC2_ROWS, DH), jnp.float32),
            pltpu.VMEM((RB * 2, CHUNK), jnp.int32),
            pltpu.VMEM((RB * 2, CHUNK), jnp.int32),
            pltpu.VMEM((CHUNK, DH), jnp.float32),
            pltpu.VMEM((CHUNK, DH), jnp.float32),
            pltpu.VMEM((CHUNK, DH), jnp.float32),
            pltpu.VMEM((CHUNK, DH), jnp.float32),
            pltpu.VMEM((OCH, DH), jnp.float32),
            pltpu.VMEM((OCH, LANES), jnp.float32),
            pltpu.VMEM((OCH, LANES), jnp.float32),
            pltpu.SemaphoreType.DMA,
            pltpu.SemaphoreType.DMA,
            [pltpu.SemaphoreType.DMA] * 4,
            [pltpu.SemaphoreType.DMA] * 4,
        ],
    )
    def k(x_hbm, idx_hbm, deg_hbm, xn_hbm,
          acc, iba, ibb, xb0, xb1, xb2, xb3, obuf, sbuf0, sbuf1,
          sema, semb, gs, ss):
        cid = lax.axis_index("c")
        sid = lax.axis_index("s")
        b0 = sid * NB
        xbs = (xb0, xb1, xb2, xb3)

        _zero_acc2(acc, obuf, sid)
        plsc.subcore_barrier()

        pltpu.async_copy(idx_hbm.at[b0], iba, sema)
        pltpu.async_copy(idx_hbm.at[b0 + 1], ibb, semb)

        @pl.loop(0, NB // 2)
        def _(j):
            blk = b0 + 2 * j
            pltpu.make_async_copy(idx_hbm.at[b0], iba, sema).wait()
            gA = [pltpu.async_copy(x_hbm.at[cid].at[iba.at[RB + r]],
                                   xbs[r], gs[r]) for r in range(RB)]
            sA = []
            for r in range(RB):
                gA[r].wait()
                sA.append(pltpu.async_copy(
                    xbs[r], acc.at[iba.at[r]], ss[r], add=True))

            pltpu.make_async_copy(idx_hbm.at[b0], ibb, semb).wait()
            gB = []
            for r in range(RB):
                sA[r].wait()
                gB.append(pltpu.async_copy(
                    x_hbm.at[cid].at[ibb.at[RB + r]], xbs[r], gs[r]))

            @pl.when(j < NB // 2 - 1)
            def _():
                pltpu.async_copy(idx_hbm.at[blk + 2], iba, sema)
            sB = []
            for r in range(RB):
                gB[r].wait()
                sB.append(pltpu.async_copy(
                    xbs[r], acc.at[ibb.at[r]], ss[r], add=True))
            for h in sB:
                h.wait()

            @pl.when(j < NB // 2 - 1)
            def _():
                pltpu.async_copy(idx_hbm.at[blk + 3], ibb, semb)

        plsc.subcore_barrier()

        @pl.loop(0, NROWS_PER_TILE // OCH)
        def _(kk):
            r0 = sid * NROWS_PER_TILE + kk * OCH
            pltpu.sync_copy(acc.at[pl.ds(r0, OCH)], obuf)
            pltpu.sync_copy(deg_hbm.at[0].at[pl.ds(r0, OCH)], sbuf0)
            pltpu.sync_copy(deg_hbm.at[1].at[pl.ds(r0, OCH)], sbuf1)

            @pl.loop(0, OCH)
            def _(i):
                d = sbuf0[i, :] + sbuf1[i, :]
                s = jnp.where(d > 0.0, 1.0 / d, 0.0)
                for j in range(DH // LANES):
                    sl = pl.ds(j * LANES, LANES)
                    obuf[i, sl] = obuf[i, sl] * s

            pltpu.sync_copy(obuf, xn_hbm.at[cid].at[pl.ds(r0, OCH)])

    return k(xs, idxc, degp)


def _zero_acc2(acc, obuf, sid):
    @pl.loop(0, OCH)
    def _(i):
        for j in range(DH // LANES):
            obuf[i, pl.ds(j * LANES, LANES)] = jnp.zeros((LANES,), jnp.float32)

    @pl.loop(0, NROWS_PER_TILE // OCH)
    def _(j):
        pltpu.sync_copy(
            obuf, acc.at[pl.ds(sid * NROWS_PER_TILE + j * OCH, OCH)])


def _mean4_kernel(a_ref, b_ref, c_ref, d_ref, o_ref):
    o_ref[...] = (a_ref[...] + b_ref[...] + c_ref[...] + d_ref[...]) * 0.25


@jax.jit
def _mean4(a, b, c, d):
    blk = 3136
    spec = pl.BlockSpec((1, blk, DH), lambda i, j: (i, j, 0))
    return pl.pallas_call(
        _mean4_kernel,
        grid=(2, NPAD // blk),
        in_specs=[spec, spec, spec, spec],
        out_specs=spec,
        out_shape=jax.ShapeDtypeStruct((2, NPAD, DH), jnp.float32),
    )(a, b, c, d)


@jax.jit
def kernel(edge_index, weight):
    row = edge_index[0].astype(jnp.int32)
    col = edge_index[1].astype(jnp.int32)
    rowp = jnp.concatenate([row, jnp.full((EPAD - E,), DUMMY2, jnp.int32)])
    colp = jnp.concatenate([col, jnp.zeros((EPAD - E,), jnp.int32)])
    idxc = jnp.concatenate(
        [rowp.reshape(NBLK, RB, CHUNK), colp.reshape(NBLK, RB, CHUNK)], axis=1)
    pad = jnp.zeros((NPAD - N, DH), jnp.float32)
    x0 = jnp.stack([jnp.concatenate([weight[:, :DH], pad]),
                    jnp.concatenate([weight[:, DH:], pad])])

    degp = _deg_partial(idxc)
    x1 = _layer(x0, idxc, degp)
    x2 = _layer(x1, idxc, degp)
    x3 = _layer(x2, idxc, degp)
    m = _mean4(x0, x1, x2, x3)
    return jnp.concatenate([m[0], m[1]], axis=1)[:N]


# speedup vs baseline: 1.2198x; 1.0030x over previous
import functools

import jax
import jax.numpy as jnp
from jax import lax
from jax.experimental import pallas as pl
from jax.experimental.pallas import tpu as pltpu
from jax.experimental.pallas import tpu_sc as plsc

N = 50000
D = 64
E = 800000
NUM_LAYERS = 3

NC, NS, LANES = 2, 16, 16

NPAD = 50176

CHUNK = 128
BLK = 512
RB = BLK // CHUNK
NB = 98
NBLK = NS * NB
EPAD = NBLK * BLK

OCH = 112

_mesh = plsc.VectorSubcoreMesh(
    core_axis_name="c", subcore_axis_name="s", num_cores=NC, num_subcores=NS
)
_sc_params = pltpu.CompilerParams(use_tc_tiling_on_sc=False)


@jax.jit
def _deg_partial(idxc):
    @functools.partial(
        pl.kernel,
        out_type=jax.ShapeDtypeStruct((2, NPAD, LANES), jnp.float32),
        mesh=_mesh,
        compiler_params=_sc_params,
        scratch_types=[
            pltpu.VMEM_SHARED((ACC2_ROWS, LANES), jnp.float32),
            pltpu.VMEM((RB * 2, CHUNK), jnp.int32),
            pltpu.VMEM((RB * 2, CHUNK), jnp.int32),
            pltpu.VMEM((CHUNK, LANES), jnp.float32),
            pltpu.VMEM((OCH, LANES), jnp.float32),
            pltpu.SemaphoreType.DMA,
            pltpu.SemaphoreType.DMA,
            pltpu.SemaphoreType.DMA,
        ],
    )
    def k(idx_hbm, deg_hbm, acc, iba, ibb, ones, obuf, sema, semb, sems):
        cid = lax.axis_index("c")
        sid = lax.axis_index("s")
        b0 = sid * NB

        @pl.loop(0, CHUNK)
        def _(i):
            ones[i, :] = jnp.ones((LANES,), jnp.float32)

        @pl.loop(0, OCH)
        def _(i):
            obuf[i, :] = jnp.zeros((LANES,), jnp.float32)

        @pl.loop(0, NROWS_PER_TILE // OCH)
        def _(j):
            pltpu.async_copy(
                obuf, acc.at[pl.ds(sid * NROWS_PER_TILE + j * OCH, OCH)],
                sems)

        @pl.loop(0, NROWS_PER_TILE // OCH)
        def _(j):
            pltpu.make_async_copy(
                obuf, acc.at[pl.ds(sid * NROWS_PER_TILE + j * OCH, OCH)],
                sems).wait()

        plsc.subcore_barrier()

        pltpu.async_copy(idx_hbm.at[b0], iba, sema)
        pltpu.async_copy(idx_hbm.at[b0 + 1], ibb, semb)

        @pl.loop(0, NB // 2)
        def _(j):
            blk = b0 + 2 * j
            for ib, sem, off in ((iba, sema, 2), (ibb, semb, 3)):
                pltpu.make_async_copy(idx_hbm.at[b0], ib, sem).wait()
                hs = [pltpu.async_copy(
                          ones, acc.at[ib.at[2 * cid + r]], sems, add=True)
                      for r in range(2)]
                for h in hs:
                    h.wait()

                @pl.when(j < NB // 2 - 1)
                def _():
                    pltpu.async_copy(idx_hbm.at[blk + off], ib, sem)

        plsc.subcore_barrier()

        r0 = sid * NROWS_PER_TILE
        pltpu.sync_copy(acc.at[pl.ds(r0, NROWS_PER_TILE)],
                        deg_hbm.at[cid].at[pl.ds(r0, NROWS_PER_TILE)])

    return k(idxc)


DH = D // 2
ACC2_ROWS = NPAD + 8
DUMMY2 = NPAD
NROWS_PER_TILE = NPAD // NS


@jax.jit
def _layer(xs, idxc, degp):
    @functools.partial(
        pl.kernel,
        out_type=jax.ShapeDtypeStruct((2, NPAD, DH), jnp.float32),
        mesh=_mesh,
        compiler_params=_sc_params,
        scratch_types=[
            pltpu.VMEM_SHARED((ACC2_ROWS, DH), jnp.float32),
            pltpu.VMEM((RB * 2, CHUNK), jnp.int32),
            pltpu.VMEM((RB * 2, CHUNK), jnp.int32),
            pltpu.VMEM((CHUNK, DH), jnp.float32),
            pltpu.VMEM((CHUNK, DH), jnp.float32),
            pltpu.VMEM((CHUNK, DH), jnp.float32),
            pltpu.VMEM((CHUNK, DH), jnp.float32),
            pltpu.VMEM((OCH, DH), jnp.float32),
            pltpu.VMEM((OCH, LANES), jnp.float32),
            pltpu.VMEM((OCH, LANES), jnp.float32),
            pltpu.SemaphoreType.DMA,
            pltpu.SemaphoreType.DMA,
            [pltpu.SemaphoreType.DMA] * 4,
            [pltpu.SemaphoreType.DMA] * 4,
        ],
    )
    def k(x_hbm, idx_hbm, deg_hbm, xn_hbm,
          acc, iba, ibb, xb0, xb1, xb2, xb3, obuf, sbuf0, sbuf1,
          sema, semb, gs, ss):
        cid = lax.axis_index("c")
        sid = lax.axis_index("s")
        b0 = sid * NB
        xbs = (xb0, xb1, xb2, xb3)

        _zero_acc2(acc, obuf, sid, sema)
        plsc.subcore_barrier()

        pltpu.async_copy(idx_hbm.at[b0], iba, sema)
        pltpu.async_copy(idx_hbm.at[b0 + 1], ibb, semb)

        @pl.loop(0, NB // 2)
        def _(j):
            blk = b0 + 2 * j
            pltpu.make_async_copy(idx_hbm.at[b0], iba, sema).wait()
            gA = [pltpu.async_copy(x_hbm.at[cid].at[iba.at[RB + r]],
                                   xbs[r], gs[r]) for r in range(RB)]
            sA = []
            for r in range(RB):
                gA[r].wait()
                sA.append(pltpu.async_copy(
                    xbs[r], acc.at[iba.at[r]], ss[r], add=True))

            pltpu.make_async_copy(idx_hbm.at[b0], ibb, semb).wait()
            gB = []
            for r in range(RB):
                sA[r].wait()
                gB.append(pltpu.async_copy(
                    x_hbm.at[cid].at[ibb.at[RB + r]], xbs[r], gs[r]))

            @pl.when(j < NB // 2 - 1)
            def _():
                pltpu.async_copy(idx_hbm.at[blk + 2], iba, sema)
            sB = []
            for r in range(RB):
                gB[r].wait()
                sB.append(pltpu.async_copy(
                    xbs[r], acc.at[ibb.at[r]], ss[r], add=True))
            for h in sB:
                h.wait()

            @pl.when(j < NB // 2 - 1)
            def _():
                pltpu.async_copy(idx_hbm.at[blk + 3], ibb, semb)

        plsc.subcore_barrier()

        @pl.loop(0, NROWS_PER_TILE // OCH)
        def _(kk):
            r0 = sid * NROWS_PER_TILE + kk * OCH
            pltpu.sync_copy(acc.at[pl.ds(r0, OCH)], obuf)
            pltpu.sync_copy(deg_hbm.at[0].at[pl.ds(r0, OCH)], sbuf0)
            pltpu.sync_copy(deg_hbm.at[1].at[pl.ds(r0, OCH)], sbuf1)

            @pl.loop(0, OCH)
            def _(i):
                d = sbuf0[i, :] + sbuf1[i, :]
                s = jnp.where(d > 0.0, 1.0 / d, 0.0)
                for j in range(DH // LANES):
                    sl = pl.ds(j * LANES, LANES)
                    obuf[i, sl] = obuf[i, sl] * s

            pltpu.sync_copy(obuf, xn_hbm.at[cid].at[pl.ds(r0, OCH)])

    return k(xs, idxc, degp)


def _zero_acc2(acc, obuf, sid, sem):
    @pl.loop(0, OCH)
    def _(i):
        for j in range(DH // LANES):
            obuf[i, pl.ds(j * LANES, LANES)] = jnp.zeros((LANES,), jnp.float32)

    @pl.loop(0, NROWS_PER_TILE // OCH)
    def _(j):
        pltpu.async_copy(
            obuf, acc.at[pl.ds(sid * NROWS_PER_TILE + j * OCH, OCH)], sem)

    @pl.loop(0, NROWS_PER_TILE // OCH)
    def _(j):
        pltpu.make_async_copy(
            obuf, acc.at[pl.ds(sid * NROWS_PER_TILE + j * OCH, OCH)],
            sem).wait()


def _mean4_kernel(a_ref, b_ref, c_ref, d_ref, o_ref):
    o_ref[...] = (a_ref[...] + b_ref[...] + c_ref[...] + d_ref[...]) * 0.25


@jax.jit
def _mean4(a, b, c, d):
    blk = 3136
    spec = pl.BlockSpec((1, blk, DH), lambda i, j: (i, j, 0))
    return pl.pallas_call(
        _mean4_kernel,
        grid=(2, NPAD // blk),
        in_specs=[spec, spec, spec, spec],
        out_specs=spec,
        out_shape=jax.ShapeDtypeStruct((2, NPAD, DH), jnp.float32),
    )(a, b, c, d)


@jax.jit
def kernel(edge_index, weight):
    row = edge_index[0].astype(jnp.int32)
    col = edge_index[1].astype(jnp.int32)
    rowp = jnp.concatenate([row, jnp.full((EPAD - E,), DUMMY2, jnp.int32)])
    colp = jnp.concatenate([col, jnp.zeros((EPAD - E,), jnp.int32)])
    idxc = jnp.concatenate(
        [rowp.reshape(NBLK, RB, CHUNK), colp.reshape(NBLK, RB, CHUNK)], axis=1)
    pad = jnp.zeros((NPAD - N, DH), jnp.float32)
    x0 = jnp.stack([jnp.concatenate([weight[:, :DH], pad]),
                    jnp.concatenate([weight[:, DH:], pad])])

    degp = _deg_partial(idxc)
    x1 = _layer(x0, idxc, degp)
    x2 = _layer(x1, idxc, degp)
    x3 = _layer(x2, idxc, degp)
    m = _mean4(x0, x1, x2, x3)
    return jnp.concatenate([m[0], m[1]], axis=1)[:N]
